# baseline (speedup 1.0000x reference)
import functools

import jax
import jax.numpy as jnp
from jax import lax
from jax.experimental import pallas as pl
from jax.experimental.pallas import tpu as pltpu
from jax.experimental.pallas import tpu_sc as plsc

DIM = 64
VOCAB = 100
VOCAB_PAD = 104
BATCH = 16384
HIST = 50
HIST_PAD = 64
NC, NS = 2, 16
NW = NC * NS
BPW = BATCH // NW
LANE = 16
GRP = BPW // LANE


def _table_body(k_ref, tab_ref, out_ref):
    k = k_ref[0, 0]
    kc = jnp.clip(k, 0.1, 10.0)
    sqrt_k = jnp.sqrt(kc + 1e-08)
    u = tab_ref[:, :]
    norm = jnp.sqrt(jnp.sum(u * u, axis=1, keepdims=True)) + 1e-08
    out_ref[:, :] = jnp.tanh(sqrt_k * norm) * u / (norm + 1e-08)


def _transform_table(table_pad, k_arr):
    return pl.pallas_call(
        _table_body,
        out_shape=jax.ShapeDtypeStruct((VOCAB_PAD, DIM), jnp.float32),
        in_specs=[
            pl.BlockSpec(memory_space=pltpu.SMEM),
            pl.BlockSpec(memory_space=pltpu.VMEM),
        ],
        out_specs=pl.BlockSpec(memory_space=pltpu.VMEM),
    )(k_arr, table_pad)


_MESH = plsc.VectorSubcoreMesh(
    core_axis_name="c", subcore_axis_name="s", num_cores=NC, num_subcores=NS
)


@functools.partial(
    pl.kernel,
    mesh=_MESH,
    compiler_params=pltpu.CompilerParams(needs_layout_passes=False),
    out_type=jax.ShapeDtypeStruct((HIST, DIM, BATCH), jnp.float32),
    scratch_types=[
        pltpu.VMEM((VOCAB_PAD * DIM,), jnp.float32),
        pltpu.VMEM((HIST_PAD, BPW), jnp.int32),
        pltpu.VMEM((2, DIM, BPW), jnp.float32),
        pltpu.SemaphoreType.DMA((2,)),
    ],
)
def _gather(ttable_hbm, idx_hbm, out_hbm, ttab_v, idx_v, stg, wsem):
    wid = lax.axis_index("s") * NC + lax.axis_index("c")
    b0 = wid * BPW
    pltpu.sync_copy(ttable_hbm, ttab_v)
    pltpu.sync_copy(idx_hbm.at[:, pl.ds(b0, BPW)], idx_v)

    def hstep(h, carry):
        def grp(g, c):
            bv = idx_v[h, pl.ds(LANE * g, LANE)]
            base = bv * DIM
            for d in range(DIM):
                vals = plsc.load_gather(ttab_v, [base + d])
                stg[0, d, pl.ds(LANE * g, LANE)] = vals
            return c

        lax.fori_loop(0, GRP, grp, 0)
        pltpu.async_copy(
            stg.at[0], out_hbm.at[h, :, pl.ds(b0, BPW)], wsem.at[0]
        ).wait()
        return carry

    lax.fori_loop(0, HIST, hstep, 0)


def kernel(idx, table, K):
    table_pad = jnp.pad(table, ((0, VOCAB_PAD - VOCAB), (0, 0)))
    ttable = _transform_table(table_pad, K.reshape(1, 1)).reshape(VOCAB_PAD * DIM)
    idx_t = jnp.pad(idx.astype(jnp.int32).T, ((0, HIST_PAD - HIST), (0, 0)))
    out = _gather(ttable, idx_t)
    return jnp.transpose(out, (2, 0, 1))


# speedup vs baseline: 1.7563x; 1.7563x over previous
import functools

import jax
import jax.numpy as jnp
from jax import lax
from jax.experimental import pallas as pl
from jax.experimental.pallas import tpu as pltpu
from jax.experimental.pallas import tpu_sc as plsc

DIM = 64
VOCAB = 100
VOCAB_PAD = 104
BATCH = 16384
HIST = 50
HIST_PAD = 64
NC, NS = 2, 16
NW = NC * NS
BPW = BATCH // NW
LANE = 16
GRP = BPW // LANE


def _table_body(k_ref, tab_ref, out_ref):
    k = k_ref[0, 0]
    kc = jnp.clip(k, 0.1, 10.0)
    sqrt_k = jnp.sqrt(kc + 1e-08)
    u = tab_ref[:, :]
    norm = jnp.sqrt(jnp.sum(u * u, axis=1, keepdims=True)) + 1e-08
    out_ref[:, :] = jnp.tanh(sqrt_k * norm) * u / (norm + 1e-08)


def _transform_table(table_pad, k_arr):
    return pl.pallas_call(
        _table_body,
        out_shape=jax.ShapeDtypeStruct((VOCAB_PAD, DIM), jnp.float32),
        in_specs=[
            pl.BlockSpec(memory_space=pltpu.SMEM),
            pl.BlockSpec(memory_space=pltpu.VMEM),
        ],
        out_specs=pl.BlockSpec(memory_space=pltpu.VMEM),
    )(k_arr, table_pad)


_MESH = plsc.VectorSubcoreMesh(
    core_axis_name="c", subcore_axis_name="s", num_cores=NC, num_subcores=NS
)


@functools.partial(
    pl.kernel,
    mesh=_MESH,
    compiler_params=pltpu.CompilerParams(needs_layout_passes=False),
    out_type=jax.ShapeDtypeStruct((HIST, DIM, BATCH), jnp.float32),
    scratch_types=[
        pltpu.VMEM((VOCAB_PAD * DIM,), jnp.float32),
        pltpu.VMEM((HIST_PAD, BPW), jnp.int32),
        pltpu.VMEM((2, DIM, BPW), jnp.float32),
        pltpu.SemaphoreType.DMA((2,)),
    ],
)
def _gather(ttable_hbm, idx_hbm, out_hbm, ttab_v, idx_v, stg, wsem):
    wid = lax.axis_index("s") * NC + lax.axis_index("c")
    b0 = wid * BPW
    pltpu.sync_copy(ttable_hbm, ttab_v)
    pltpu.sync_copy(idx_hbm.at[:, pl.ds(b0, BPW)], idx_v)

    def pair(hp, carry):
        for q in (0, 1):
            h = 2 * hp + q
            dst = out_hbm.at[h, :, pl.ds(b0, BPW)]

            @pl.when(hp > 0)
            def _():
                pltpu.make_async_copy(stg.at[q], dst, wsem.at[q]).wait()

            def grp(g, c):
                bv = idx_v[h, pl.ds(LANE * g, LANE)]
                base = bv * DIM
                for dblk in range(0, DIM, 8):
                    vals = [
                        plsc.load_gather(ttab_v, [base + (dblk + k)])
                        for k in range(8)
                    ]
                    for k in range(8):
                        stg[q, dblk + k, pl.ds(LANE * g, LANE)] = vals[k]
                return c

            lax.fori_loop(0, GRP, grp, 0)
            pltpu.async_copy(stg.at[q], dst, wsem.at[q])
        return carry

    lax.fori_loop(0, HIST // 2, pair, 0)
    for q in (0, 1):
        pltpu.make_async_copy(
            stg.at[q], out_hbm.at[HIST - 2 + q, :, pl.ds(b0, BPW)], wsem.at[q]
        ).wait()


def kernel(idx, table, K):
    table_pad = jnp.pad(table, ((0, VOCAB_PAD - VOCAB), (0, 0)))
    ttable = _transform_table(table_pad, K.reshape(1, 1)).reshape(VOCAB_PAD * DIM)
    idx_t = jnp.pad(idx.astype(jnp.int32).T, ((0, HIST_PAD - HIST), (0, 0)))
    out = _gather(ttable, idx_t)
    return jnp.transpose(out, (2, 0, 1))


# speedup vs baseline: 8.5870x; 4.8892x over previous
import functools

import jax
import jax.numpy as jnp
from jax import lax
from jax.experimental import pallas as pl
from jax.experimental.pallas import tpu as pltpu
from jax.experimental.pallas import tpu_sc as plsc

DIM = 64
VOCAB = 100
VOCAB_PAD = 104
BATCH = 16384
HIST = 50
HIST_PAD = 64
NC, NS = 2, 16
NW = NC * NS
BPW = BATCH // NW
LANE = 16
GRP = BPW // LANE


def _table_body(k_ref, tab_ref, out_ref):
    k = k_ref[0, 0]
    kc = jnp.clip(k, 0.1, 10.0)
    sqrt_k = jnp.sqrt(kc + 1e-08)
    u = tab_ref[:, :]
    norm = jnp.sqrt(jnp.sum(u * u, axis=1, keepdims=True)) + 1e-08
    out_ref[:, :] = jnp.tanh(sqrt_k * norm) * u / (norm + 1e-08)


def _transform_table(table_pad, k_arr):
    return pl.pallas_call(
        _table_body,
        out_shape=jax.ShapeDtypeStruct((VOCAB_PAD, DIM), jnp.float32),
        in_specs=[
            pl.BlockSpec(memory_space=pltpu.SMEM),
            pl.BlockSpec(memory_space=pltpu.VMEM),
        ],
        out_specs=pl.BlockSpec(memory_space=pltpu.VMEM),
    )(k_arr, table_pad)


_MESH = plsc.VectorSubcoreMesh(
    core_axis_name="c", subcore_axis_name="s", num_cores=NC, num_subcores=NS
)


@functools.partial(
    pl.kernel,
    mesh=_MESH,
    compiler_params=pltpu.CompilerParams(needs_layout_passes=False),
    out_type=jax.ShapeDtypeStruct((HIST, DIM, BATCH), jnp.float32),
    scratch_types=[
        pltpu.VMEM((VOCAB_PAD * DIM,), jnp.float32),
        pltpu.VMEM((HIST_PAD, BPW), jnp.int32),
        pltpu.VMEM((2, DIM, BPW), jnp.float32),
        pltpu.SemaphoreType.DMA((2,)),
    ],
)
def _gather(ttable_hbm, idx_hbm, out_hbm, ttab_v, idx_v, stg, wsem):
    wid = lax.axis_index("s") * NC + lax.axis_index("c")
    b0 = wid * BPW
    pltpu.sync_copy(ttable_hbm, ttab_v)
    pltpu.sync_copy(idx_hbm.at[:, pl.ds(b0, BPW)], idx_v)

    def pair(hp, carry):
        for q in (0, 1):
            h = 2 * hp + q
            dst = out_hbm.at[h, :, pl.ds(b0, BPW)]

            @pl.when(hp > 0)
            def _():
                pltpu.make_async_copy(stg.at[q], dst, wsem.at[q]).wait()

            def grp(g, c):
                bv = idx_v[h, pl.ds(LANE * g, LANE)]
                for dblk in range(0, DIM, 8):
                    vals = [
                        plsc.load_gather(ttab_v, [bv + VOCAB_PAD * (dblk + k)])
                        for k in range(8)
                    ]
                    for k in range(8):
                        stg[q, dblk + k, pl.ds(LANE * g, LANE)] = vals[k]
                return c

            lax.fori_loop(0, GRP, grp, 0)
            pltpu.async_copy(stg.at[q], dst, wsem.at[q])
        return carry

    lax.fori_loop(0, HIST // 2, pair, 0)
    for q in (0, 1):
        pltpu.make_async_copy(
            stg.at[q], out_hbm.at[HIST - 2 + q, :, pl.ds(b0, BPW)], wsem.at[q]
        ).wait()


def kernel(idx, table, K):
    table_pad = jnp.pad(table, ((0, VOCAB_PAD - VOCAB), (0, 0)))
    ttable = _transform_table(table_pad, K.reshape(1, 1)).T.reshape(VOCAB_PAD * DIM)
    idx_t = jnp.pad(idx.astype(jnp.int32).T, ((0, HIST_PAD - HIST), (0, 0)))
    out = _gather(ttable, idx_t)
    return jnp.transpose(out, (2, 0, 1))
